# Initial kernel scaffold; baseline (speedup 1.0000x reference)
import jax
import jax.numpy as jnp
from jax.experimental import pallas as pl


def kernel(x, edge_index, edge_weight):
    raise NotImplementedError("write your pallas kernel here")


# speedup vs baseline: 10.6368x; 10.6368x over previous
import functools

import jax
import jax.numpy as jnp
from jax import lax
from jax.experimental import pallas as pl
from jax.experimental.pallas import tpu as pltpu
from jax.experimental.pallas import tpu_sc as plsc

NC = 2
NS = 16
L = 16
B = 80


def _rsqrt_nr(d):
  di = plsc.bitcast(d, jnp.int32)
  y = plsc.bitcast(jnp.int32(0x5F3759DF) - (di >> 1), jnp.float32)
  for _ in range(3):
    y = y * (1.5 - 0.5 * d * y * y)
  return jnp.where(d > 0.0, y, 0.0)


def _make_sc_call(N, D, E):
  NPAD = ((N + NC * NS * L - 1) // (NC * NS * L)) * NC * NS * L
  NPT = NPAD // NS
  EH = E // NS
  EPT = E // (NC * NS)
  assert E % (NC * NS) == 0 and EH % B == 0 and EPT % B == 0
  assert NPT % L == 0 and B % L == 0

  mesh = plsc.VectorSubcoreMesh(
      core_axis_name="c", subcore_axis_name="s", num_cores=NC,
      num_subcores=NS)

  def body(x_hbm, row_hbm, col_hbm, w_hbm, part_hbm,
           d_sh, s_sh, out_sh, s_loc, dbuf, sbuf, rowb, colb, wb, valb,
           xrows):
    c = lax.axis_index("c")
    tid = lax.axis_index("s")
    wid = c * NS + tid

    def zrow(b, _):
      for j in range(D // L):
        xrows[b, pl.ds(j * L, L)] = jnp.zeros((L,), jnp.float32)
      return 0
    lax.fori_loop(0, B, zrow, 0)

    def zvec(i, _):
      dbuf[pl.ds(i * L, L)] = jnp.zeros((L,), jnp.float32)
      return 0
    lax.fori_loop(0, NPT // L, zvec, 0)
    pltpu.sync_copy(dbuf, d_sh.at[pl.ds(tid * NPT, NPT)])
    for k in range(NPT // B):
      pltpu.sync_copy(xrows, out_sh.at[pl.ds(tid * NPT + k * B, B)])

    def ones(i, _):
      valb[pl.ds(i * L, L)] = jnp.full((L,), 1.0, jnp.float32)
      return 0
    lax.fori_loop(0, B // L, ones, 0)
    plsc.subcore_barrier()

    def hist(i, _):
      pltpu.sync_copy(col_hbm.at[pl.ds(tid * EH + i * B, B)], colb)
      pltpu.sync_copy(valb, d_sh.at[colb], add=True)
      return 0
    lax.fori_loop(0, EH // B, hist, 0)
    plsc.subcore_barrier()

    pltpu.sync_copy(d_sh.at[pl.ds(tid * NPT, NPT)], dbuf)

    def rs(i, _):
      sbuf[pl.ds(i * L, L)] = _rsqrt_nr(dbuf[pl.ds(i * L, L)])
      return 0
    lax.fori_loop(0, NPT // L, rs, 0)
    pltpu.sync_copy(sbuf, s_sh.at[pl.ds(tid * NPT, NPT)])
    plsc.subcore_barrier()

    pltpu.sync_copy(s_sh, s_loc)

    def batch(i, _):
      off = wid * EPT + i * B
      pltpu.sync_copy(row_hbm.at[pl.ds(off, B)], rowb)
      pltpu.sync_copy(col_hbm.at[pl.ds(off, B)], colb)
      pltpu.sync_copy(w_hbm.at[pl.ds(off, B)], wb)
      pltpu.sync_copy(x_hbm.at[rowb], xrows)

      for k in range(B // L):
        ri = rowb[pl.ds(k * L, L)]
        ci = colb[pl.ds(k * L, L)]
        sv = (plsc.load_gather(s_loc, [ci]) * plsc.load_gather(s_loc, [ri])
              * wb[pl.ds(k * L, L)])
        valb[pl.ds(k * L, L)] = sv

      def scale(k, _):
        vv = valb[pl.ds(k * L, L)]
        for i in range(L):
          v = vv[i]
          b = k * L + i
          for j in range(D // L):
            xrows[b, pl.ds(j * L, L)] = xrows[b, pl.ds(j * L, L)] * v
        return 0
      lax.fori_loop(0, B // L, scale, 0)

      pltpu.sync_copy(xrows, out_sh.at[colb], add=True)
      return 0
    lax.fori_loop(0, EPT // B, batch, 0)
    plsc.subcore_barrier()

    pltpu.sync_copy(out_sh.at[pl.ds(tid * NPT, NPT)],
                    part_hbm.at[c, pl.ds(tid * NPT, NPT)])

  return pl.kernel(
      body,
      out_type=jax.ShapeDtypeStruct((NC, NPAD, D), jnp.float32),
      mesh=mesh,
      compiler_params=pltpu.CompilerParams(needs_layout_passes=False),
      scratch_types=[
          pltpu.VMEM_SHARED((NPAD,), jnp.float32),
          pltpu.VMEM_SHARED((NPAD,), jnp.float32),
          pltpu.VMEM_SHARED((NPAD, D), jnp.float32),
          pltpu.VMEM((NPAD,), jnp.float32),
          pltpu.VMEM((NPT,), jnp.float32),
          pltpu.VMEM((NPT,), jnp.float32),
          pltpu.VMEM((B,), jnp.int32),
          pltpu.VMEM((B,), jnp.int32),
          pltpu.VMEM((B,), jnp.float32),
          pltpu.VMEM((B,), jnp.float32),
          pltpu.VMEM((B, D), jnp.float32),
      ],
  )


def _sum_body(p_ref, o_ref):
  o_ref[...] = p_ref[0] + p_ref[1]


@jax.jit
def kernel(x, edge_index, edge_weight):
  N, D = x.shape
  E = edge_weight.shape[0]
  row = edge_index[0].astype(jnp.int32)
  col = edge_index[1].astype(jnp.int32)
  sc_call = _make_sc_call(N, D, E)
  partials = sc_call(x, row, col, edge_weight)

  NPAD = partials.shape[1]
  grid = 10
  rb = NPAD // grid
  out = pl.pallas_call(
      _sum_body,
      grid=(grid,),
      in_specs=[pl.BlockSpec((NC, rb, D), lambda i: (0, i, 0))],
      out_specs=pl.BlockSpec((rb, D), lambda i: (i, 0)),
      out_shape=jax.ShapeDtypeStruct((NPAD, D), jnp.float32),
  )(partials)
  return out[:N]


# speedup vs baseline: 21.2625x; 1.9990x over previous
import functools

import jax
import jax.numpy as jnp
from jax import lax
from jax.experimental import pallas as pl
from jax.experimental.pallas import tpu as pltpu
from jax.experimental.pallas import tpu_sc as plsc

NC = 2
NS = 16
L = 16
B = 80


def _rsqrt_nr(d):
  di = plsc.bitcast(d, jnp.int32)
  y = plsc.bitcast(jnp.int32(0x5F3759DF) - (di >> 1), jnp.float32)
  for _ in range(3):
    y = y * (1.5 - 0.5 * d * y * y)
  return jnp.where(d > 0.0, y, 0.0)


def _make_sc_call(N, D, E):
  NPAD = ((N + NC * NS * L - 1) // (NC * NS * L)) * NC * NS * L
  NPT = NPAD // NS
  EH = E // NS
  EPT = E // (NC * NS)
  NB = EPT // B
  NH = EH // B
  assert E % (NC * NS) == 0 and EH % B == 0 and EPT % B == 0
  assert NPT % L == 0 and B % L == 0 and NH % 2 == 0 and NB % 2 == 1

  mesh = plsc.VectorSubcoreMesh(
      core_axis_name="c", subcore_axis_name="s", num_cores=NC,
      num_subcores=NS)

  def body(x_hbm, row_hbm, col_hbm, w_hbm, part_hbm,
           d_sh, s_sh, out_sh, s_loc, dbuf, sbuf, rowb, colb, wb, valb,
           xrows, semi0, semi1, semx0, semx1):
    c = lax.axis_index("c")
    tid = lax.axis_index("s")
    wid = c * NS + tid
    semi = (semi0, semi1)
    semx = (semx0, semx1)

    def zrow(b, _):
      for j in range(D // L):
        xrows[0, b, pl.ds(j * L, L)] = jnp.zeros((L,), jnp.float32)
      return 0
    lax.fori_loop(0, B, zrow, 0)

    def zvec(i, _):
      dbuf[pl.ds(i * L, L)] = jnp.zeros((L,), jnp.float32)
      return 0
    lax.fori_loop(0, NPT // L, zvec, 0)
    pltpu.sync_copy(dbuf, d_sh.at[pl.ds(tid * NPT, NPT)])
    for k in range(NPT // B):
      pltpu.sync_copy(xrows.at[0], out_sh.at[pl.ds(tid * NPT + k * B, B)])

    def ones(i, _):
      valb[0, pl.ds(i * L, L)] = jnp.full((L,), 1.0, jnp.float32)
      return 0
    lax.fori_loop(0, B // L, ones, 0)
    plsc.subcore_barrier()

    def h_issue(i, p):
      pltpu.async_copy(col_hbm.at[pl.ds(tid * EH + i * B, B)],
                       colb.at[p], semi[p])

    def h_wait(p):
      pltpu.make_async_copy(col_hbm.at[pl.ds(0, B)], colb.at[p],
                            semi[p]).wait()

    h_issue(0, 0)
    h_issue(1, 1)

    def hist(k, _):
      for p in range(2):
        i = 2 * k + p
        h_wait(p)
        pltpu.sync_copy(valb.at[0], d_sh.at[colb.at[p]], add=True)

        @pl.when(k < NH // 2 - 1)
        def _():
          h_issue(i + 2, p)
      return 0
    lax.fori_loop(0, NH // 2, hist, 0)
    plsc.subcore_barrier()

    pltpu.sync_copy(d_sh.at[pl.ds(tid * NPT, NPT)], dbuf)

    def rs(i, _):
      sbuf[pl.ds(i * L, L)] = _rsqrt_nr(dbuf[pl.ds(i * L, L)])
      return 0
    lax.fori_loop(0, NPT // L, rs, 0)
    pltpu.sync_copy(sbuf, s_sh.at[pl.ds(tid * NPT, NPT)])
    plsc.subcore_barrier()

    pltpu.sync_copy(s_sh, s_loc)

    def e_issue(i, p):
      off = wid * EPT + i * B
      pltpu.async_copy(row_hbm.at[pl.ds(off, B)], rowb.at[p], semi[p])
      pltpu.async_copy(col_hbm.at[pl.ds(off, B)], colb.at[p], semi[p])
      pltpu.async_copy(w_hbm.at[pl.ds(off, B)], wb.at[p], semi[p])

    def e_wait(p):
      pltpu.make_async_copy(row_hbm.at[pl.ds(0, B)], rowb.at[p],
                            semi[p]).wait()
      pltpu.make_async_copy(col_hbm.at[pl.ds(0, B)], colb.at[p],
                            semi[p]).wait()
      pltpu.make_async_copy(w_hbm.at[pl.ds(0, B)], wb.at[p],
                            semi[p]).wait()

    def g_issue(p):
      pltpu.async_copy(x_hbm.at[rowb.at[p]], xrows.at[p], semx[p])

    def g_wait(p):
      pltpu.make_async_copy(x_hbm.at[rowb.at[p]], xrows.at[p],
                            semx[p]).wait()

    def work(i, p, last=False):
      q = 1 - p
      g_wait(p)
      if not last:
        e_wait(q)
        g_issue(q)

      for kk in range(B // L):
        ri = rowb[p, pl.ds(kk * L, L)]
        ci = colb[p, pl.ds(kk * L, L)]
        sv = (plsc.load_gather(s_loc, [ci]) * plsc.load_gather(s_loc, [ri])
              * wb[p, pl.ds(kk * L, L)])
        valb[p, pl.ds(kk * L, L)] = sv

      def scale(kk, _):
        vv = valb[p, pl.ds(kk * L, L)]
        for ii in range(L):
          v = vv[ii]
          b = kk * L + ii
          for j in range(D // L):
            xrows[p, b, pl.ds(j * L, L)] = xrows[p, b, pl.ds(j * L, L)] * v
        return 0
      lax.fori_loop(0, B // L, scale, 0)

      pltpu.sync_copy(xrows.at[p], out_sh.at[colb.at[p]], add=True)

      if not last:
        @pl.when(i + 2 < NB)
        def _():
          e_issue(i + 2, p)

    e_issue(0, 0)
    e_issue(1, 1)
    e_wait(0)
    g_issue(0)

    def batch2(k, _):
      work(2 * k, 0)
      work(2 * k + 1, 1)
      return 0
    lax.fori_loop(0, NB // 2, batch2, 0)
    work(NB - 1, (NB - 1) % 2, last=True)
    plsc.subcore_barrier()

    pltpu.sync_copy(out_sh.at[pl.ds(tid * NPT, NPT)],
                    part_hbm.at[c, pl.ds(tid * NPT, NPT)])

  return pl.kernel(
      body,
      out_type=jax.ShapeDtypeStruct((NC, NPAD, D), jnp.float32),
      mesh=mesh,
      compiler_params=pltpu.CompilerParams(needs_layout_passes=False),
      scratch_types=[
          pltpu.VMEM_SHARED((NPAD,), jnp.float32),
          pltpu.VMEM_SHARED((NPAD,), jnp.float32),
          pltpu.VMEM_SHARED((NPAD, D), jnp.float32),
          pltpu.VMEM((NPAD,), jnp.float32),
          pltpu.VMEM((NPT,), jnp.float32),
          pltpu.VMEM((NPT,), jnp.float32),
          pltpu.VMEM((2, B), jnp.int32),
          pltpu.VMEM((2, B), jnp.int32),
          pltpu.VMEM((2, B), jnp.float32),
          pltpu.VMEM((2, B), jnp.float32),
          pltpu.VMEM((2, B, D), jnp.float32),
          pltpu.SemaphoreType.DMA,
          pltpu.SemaphoreType.DMA,
          pltpu.SemaphoreType.DMA,
          pltpu.SemaphoreType.DMA,
      ],
  )


def _sum_body(p_ref, o_ref):
  o_ref[...] = p_ref[0] + p_ref[1]


@jax.jit
def kernel(x, edge_index, edge_weight):
  N, D = x.shape
  E = edge_weight.shape[0]
  row = edge_index[0].astype(jnp.int32)
  col = edge_index[1].astype(jnp.int32)
  sc_call = _make_sc_call(N, D, E)
  partials = sc_call(x, row, col, edge_weight)

  NPAD = partials.shape[1]
  grid = 10
  rb = NPAD // grid
  out = pl.pallas_call(
      _sum_body,
      grid=(grid,),
      in_specs=[pl.BlockSpec((NC, rb, D), lambda i: (0, i, 0))],
      out_specs=pl.BlockSpec((rb, D), lambda i: (i, 0)),
      out_shape=jax.ShapeDtypeStruct((NPAD, D), jnp.float32),
  )(partials)
  return out[:N]
